# baseline (speedup 1.0000x reference)
import functools
import jax
import jax.numpy as jnp
from jax import lax
from jax.experimental import pallas as pl
from jax.experimental.pallas import tpu as pltpu
from jax.experimental.pallas import tpu_sc as plsc

NUM_WORKERS = 32
IDX_SPLIT = 2


def _make_sc_kernel(b, l, d):
    bpw = b // NUM_WORKERS
    half = l // IDX_SPLIT
    mesh = plsc.VectorSubcoreMesh(core_axis_name="c", subcore_axis_name="s")

    @functools.partial(
        pl.kernel,
        mesh=mesh,
        out_type=jax.ShapeDtypeStruct((b * l, d), jnp.float32),
        scratch_types=[
            pltpu.VMEM((bpw * IDX_SPLIT, half), jnp.int32),
            pltpu.VMEM_SHARED((l, d), jnp.float32),
            pltpu.VMEM((l, d), jnp.float32),
            pltpu.VMEM((l, d), jnp.float32),
        ]
        + [pltpu.SemaphoreType.DMA] * 6,
    )
    def sc_kernel(x_hbm, pos_hbm, idx_hbm, out_hbm, idx_v, pos_sh, buf0,
                  buf1, s_i0, s_i1, s_g0, s_g1, s_o0, s_o1):
        cid = lax.axis_index("c")
        sid = lax.axis_index("s")
        wid = sid * 2 + cid
        bufs = (buf0, buf1)
        s_init = (s_i0, s_i1)
        s_gadd = (s_g0, s_g1)
        s_out = (s_o0, s_o1)

        @pl.when(sid == 0)
        def _():
            pltpu.sync_copy(pos_hbm, pos_sh)

        pltpu.sync_copy(
            idx_hbm.at[pl.ds(wid * bpw * IDX_SPLIT, bpw * IDX_SPLIT)], idx_v)
        plsc.subcore_barrier()

        gadd_h = [None] * bpw
        scat_h = [None] * bpw

        def issue_scatter(j):
            gadd_h[j][0].wait()
            gadd_h[j][1].wait()
            row0 = (wid * bpw + j) * l
            scat_h[j] = pltpu.async_copy(
                bufs[j % 2], out_hbm.at[pl.ds(row0, l)], s_out[j % 2])

        for i in range(bpw):
            p = i % 2
            if i >= 2:
                scat_h[i - 2].wait()
            init_h = pltpu.async_copy(pos_sh, bufs[p], s_init[p])
            if i >= 1:
                issue_scatter(i - 1)
            init_h.wait()
            gadd_h[i] = (
                pltpu.async_copy(
                    x_hbm.at[idx_v.at[IDX_SPLIT * i]],
                    bufs[p].at[pl.ds(0, half)], s_gadd[p], add=True),
                pltpu.async_copy(
                    x_hbm.at[idx_v.at[IDX_SPLIT * i + 1]],
                    bufs[p].at[pl.ds(half, half)], s_gadd[p], add=True),
            )
        issue_scatter(bpw - 1)
        scat_h[bpw - 2].wait()
        scat_h[bpw - 1].wait()

    return sc_kernel


def kernel(x, pos_table):
    b, l, d = x.shape
    x2 = x.reshape(b * l, d)
    idx = jnp.arange(b * l, dtype=jnp.int32).reshape(
        b * IDX_SPLIT, l // IDX_SPLIT)
    out = _make_sc_kernel(b, l, d)(x2, pos_table, idx)
    return out.reshape(b, l, d)


# speedup vs baseline: 1.0020x; 1.0020x over previous
import functools
import jax
import jax.numpy as jnp
from jax import lax
from jax.experimental import pallas as pl
from jax.experimental.pallas import tpu as pltpu
from jax.experimental.pallas import tpu_sc as plsc

NUM_WORKERS = 32
NBUF = 4
IDX_SPLIT = 2


def _make_sc_kernel(b, l, d):
    bpw = b // NUM_WORKERS
    half = l // IDX_SPLIT
    mesh = plsc.VectorSubcoreMesh(core_axis_name="c", subcore_axis_name="s")

    @functools.partial(
        pl.kernel,
        mesh=mesh,
        out_type=jax.ShapeDtypeStruct((b * l, d), jnp.float32),
        scratch_types=[
            pltpu.VMEM((bpw * IDX_SPLIT, half), jnp.int32),
            pltpu.VMEM_SHARED((l, d), jnp.float32),
        ]
        + [pltpu.VMEM((l, d), jnp.float32) for _ in range(NBUF)]
        + [pltpu.SemaphoreType.DMA] * (3 * NBUF),
    )
    def sc_kernel(x_hbm, pos_hbm, idx_hbm, out_hbm, idx_v, pos_sh, *rest):
        bufs = rest[:NBUF]
        s_init = rest[NBUF:2 * NBUF]
        s_gadd = rest[2 * NBUF:3 * NBUF]
        s_out = rest[3 * NBUF:4 * NBUF]
        cid = lax.axis_index("c")
        sid = lax.axis_index("s")
        wid = sid * 2 + cid

        @pl.when(sid == 0)
        def _():
            pltpu.sync_copy(pos_hbm, pos_sh)

        pltpu.sync_copy(
            idx_hbm.at[pl.ds(wid * bpw * IDX_SPLIT, bpw * IDX_SPLIT)], idx_v)
        plsc.subcore_barrier()

        init_h = [None] * bpw
        gadd_h = [None] * bpw
        scat_h = [None] * bpw

        def issue_init(j):
            init_h[j] = pltpu.async_copy(pos_sh, bufs[j % NBUF],
                                         s_init[j % NBUF])

        def issue_scat(j):
            scat_h[j] = pltpu.async_copy(
                bufs[j % NBUF],
                out_hbm.at[pl.ds((wid * bpw + j) * l, l)],
                s_out[j % NBUF])

        issue_init(0)
        for i in range(bpw):
            p = i % NBUF
            if i + 1 < bpw:
                if i >= NBUF - 1:
                    scat_h[i - (NBUF - 1)].wait()
                issue_init(i + 1)
            if i >= 1:
                gadd_h[i - 1][0].wait()
                gadd_h[i - 1][1].wait()
                issue_scat(i - 1)
            init_h[i].wait()
            gadd_h[i] = (
                pltpu.async_copy(
                    x_hbm.at[idx_v.at[IDX_SPLIT * i]],
                    bufs[p].at[pl.ds(0, half)], s_gadd[p], add=True),
                pltpu.async_copy(
                    x_hbm.at[idx_v.at[IDX_SPLIT * i + 1]],
                    bufs[p].at[pl.ds(half, half)], s_gadd[p], add=True),
            )
        gadd_h[bpw - 1][0].wait()
        gadd_h[bpw - 1][1].wait()
        issue_scat(bpw - 1)
        for j in range(bpw - NBUF + 1, bpw):
            scat_h[j].wait()

    return sc_kernel


def kernel(x, pos_table):
    b, l, d = x.shape
    x2 = x.reshape(b * l, d)
    idx = jnp.arange(b * l, dtype=jnp.int32).reshape(
        b * IDX_SPLIT, l // IDX_SPLIT)
    out = _make_sc_kernel(b, l, d)(x2, pos_table, idx)
    return out.reshape(b, l, d)
